# Initial kernel scaffold; baseline (speedup 1.0000x reference)
import jax
import jax.numpy as jnp
from jax.experimental import pallas as pl


def kernel(x, table):
    raise NotImplementedError("write your pallas kernel here")


# speedup vs baseline: 1.4932x; 1.4932x over previous
import functools

import jax
import jax.numpy as jnp
from jax import lax
from jax.experimental import pallas as pl
from jax.experimental.pallas import tpu as pltpu
from jax.experimental.pallas import tpu_sc as plsc

VOCAB = 1000000
EMBED_DIM = 32
BATCH = 4096
SEQ = 200

B = BATCH * SEQ
NC = 2
NS = 16
NW = NC * NS
B_PER_W = B // NW
GATHER_ROWS = 128
G_PER_CHUNK = 10
CHUNK_ROWS = GATHER_ROWS * G_PER_CHUNK
CHUNKS = B_PER_W // CHUNK_ROWS
IDX_ROWS_PER_W = B_PER_W // GATHER_ROWS
NBUF = 2


def _emb_body(table_hbm, idx_hbm, out_hbm, idx_v, rows_v,
              gsem0, gsem1, wsem0, wsem1):
    wid = lax.axis_index("s") * NC + lax.axis_index("c")
    idx_row_base = wid * IDX_ROWS_PER_W
    out_row_base = wid * B_PER_W

    pltpu.sync_copy(idx_hbm.at[pl.ds(idx_row_base, IDX_ROWS_PER_W), :], idx_v)

    gsems = (gsem0, gsem1)
    wsems = (wsem0, wsem1)

    def pair_body(p, carry):
        c0 = p * NBUF
        gathers = []
        for b in range(NBUF):
            c = c0 + b
            for j in range(G_PER_CHUNK):
                gathers.append(pltpu.async_copy(
                    table_hbm.at[idx_v.at[c * G_PER_CHUNK + j]],
                    rows_v.at[b, pl.ds(j * GATHER_ROWS, GATHER_ROWS), :],
                    gsems[b]))
        writes = []
        for b in range(NBUF):
            c = c0 + b
            for j in range(G_PER_CHUNK):
                gathers[b * G_PER_CHUNK + j].wait()
            writes.append(pltpu.async_copy(
                rows_v.at[b],
                out_hbm.at[pl.ds(out_row_base + c * CHUNK_ROWS, CHUNK_ROWS), :],
                wsems[b]))
        for w in writes:
            w.wait()
        return carry

    lax.fori_loop(0, CHUNKS // NBUF, pair_body, 0)


_gather_call = pl.kernel(
    _emb_body,
    out_type=jax.ShapeDtypeStruct((B, EMBED_DIM), jnp.float32),
    mesh=plsc.VectorSubcoreMesh(core_axis_name="c", subcore_axis_name="s"),
    compiler_params=pltpu.CompilerParams(use_tc_tiling_on_sc=False),
    scratch_types=[
        pltpu.VMEM((IDX_ROWS_PER_W, GATHER_ROWS), jnp.int32),
        pltpu.VMEM((NBUF, CHUNK_ROWS, EMBED_DIM), jnp.float32),
        pltpu.SemaphoreType.DMA,
        pltpu.SemaphoreType.DMA,
        pltpu.SemaphoreType.DMA,
        pltpu.SemaphoreType.DMA,
    ],
)


def kernel(x, table):
    idx = x.reshape(B).astype(jnp.int32).reshape(B // GATHER_ROWS, GATHER_ROWS)
    out = _gather_call(table, idx)
    return out.reshape(BATCH, SEQ, EMBED_DIM)


# speedup vs baseline: 1.4933x; 1.0001x over previous
import functools

import jax
import jax.numpy as jnp
from jax import lax
from jax.experimental import pallas as pl
from jax.experimental.pallas import tpu as pltpu
from jax.experimental.pallas import tpu_sc as plsc

VOCAB = 1000000
EMBED_DIM = 32
BATCH = 4096
SEQ = 200

B = BATCH * SEQ
NC = 2
NS = 16
NW = NC * NS
B_PER_W = B // NW
CHUNK_ROWS = 1280
CHUNKS = B_PER_W // CHUNK_ROWS
NBUF = 2


def _emb_body(table_hbm, idx_hbm, out_hbm, idx_v, rows_v,
              gsem0, gsem1, wsem0, wsem1):
    wid = lax.axis_index("s") * NC + lax.axis_index("c")
    out_row_base = wid * B_PER_W

    pltpu.sync_copy(idx_hbm.at[pl.ds(out_row_base, B_PER_W)], idx_v)

    gsems = (gsem0, gsem1)
    wsems = (wsem0, wsem1)

    def pair_body(p, carry):
        c0 = p * NBUF
        gathers = []
        for b in range(NBUF):
            c = c0 + b
            gathers.append(pltpu.async_copy(
                table_hbm.at[idx_v.at[pl.ds(c * CHUNK_ROWS, CHUNK_ROWS)]],
                rows_v.at[b],
                gsems[b]))
        writes = []
        for b in range(NBUF):
            c = c0 + b
            gathers[b].wait()
            writes.append(pltpu.async_copy(
                rows_v.at[b],
                out_hbm.at[pl.ds(out_row_base + c * CHUNK_ROWS, CHUNK_ROWS), :],
                wsems[b]))
        for w in writes:
            w.wait()
        return carry

    lax.fori_loop(0, CHUNKS // NBUF, pair_body, 0)


_gather_call = pl.kernel(
    _emb_body,
    out_type=jax.ShapeDtypeStruct((B, EMBED_DIM), jnp.float32),
    mesh=plsc.VectorSubcoreMesh(core_axis_name="c", subcore_axis_name="s"),
    compiler_params=pltpu.CompilerParams(use_tc_tiling_on_sc=False),
    scratch_types=[
        pltpu.VMEM((B_PER_W,), jnp.int32),
        pltpu.VMEM((NBUF, CHUNK_ROWS, EMBED_DIM), jnp.float32),
        pltpu.SemaphoreType.DMA,
        pltpu.SemaphoreType.DMA,
        pltpu.SemaphoreType.DMA,
        pltpu.SemaphoreType.DMA,
    ],
)


def kernel(x, table):
    idx = x.reshape(B).astype(jnp.int32)
    out = _gather_call(table, idx)
    return out.reshape(BATCH, SEQ, EMBED_DIM)
